# baseline (speedup 1.0000x reference)
import functools

import jax
import jax.numpy as jnp
from jax import lax
from jax.experimental import pallas as pl
from jax.experimental.pallas import tpu as pltpu
from jax.experimental.pallas import tpu_sc as plsc

N_CATS = 8
EPS = 1e-07
N_ROWS = 1_000_000
CHUNK = 2000
NCHUNKS = N_ROWS // CHUNK
GROUPS = CHUNK // 16
NC, NS = 2, 16
NW = NC * NS
CHUNKS_PER_W = (NCHUNKS + NW - 1) // NW


def _sc_partial_hist(inputs_flat, targets):
    mesh = plsc.VectorSubcoreMesh(core_axis_name="c", subcore_axis_name="s")

    @functools.partial(
        pl.kernel,
        mesh=mesh,
        out_type=jax.ShapeDtypeStruct((NW, 1024), jnp.float32),
        compiler_params=pltpu.CompilerParams(needs_layout_passes=False),
        scratch_types=[
            pltpu.VMEM((CHUNK * N_CATS,), jnp.float32),
            pltpu.VMEM((CHUNK,), jnp.int32),
            pltpu.VMEM((1024,), jnp.float32),
        ],
    )
    def body(in_hbm, tgt_hbm, out_hbm, in_v, tgt_v, hist_v):
        w = lax.axis_index("s") * NC + lax.axis_index("c")

        def zero_body(k, carry):
            hist_v[pl.ds(k * 16, 16)] = jnp.zeros((16,), jnp.float32)
            return carry

        lax.fori_loop(0, 64, zero_body, None)

        lane = lax.iota(jnp.int32, 16)
        row_off = lane * N_CATS
        ones = jnp.ones((16,), jnp.float32)

        def chunk_body(i, carry):
            cid = w + i * NW

            @pl.when(cid < NCHUNKS)
            def _do():
                pltpu.sync_copy(
                    in_hbm.at[pl.ds(cid * CHUNK * N_CATS, CHUNK * N_CATS)], in_v)
                pltpu.sync_copy(tgt_hbm.at[pl.ds(cid * CHUNK, CHUNK)], tgt_v)

                def group_body(g, gcarry):
                    base = g * (16 * N_CATS)
                    m = plsc.load_gather(in_v, [base + row_off])
                    am = jnp.zeros((16,), jnp.int32)
                    for j in range(1, N_CATS):
                        vj = plsc.load_gather(in_v, [base + row_off + j])
                        gt = vj > m
                        m = jnp.where(gt, vj, m)
                        am = jnp.where(gt, jnp.int32(j), am)
                    t = tgt_v[pl.ds(g * 16, 16)]
                    addr = t * 128 + am * 16 + lane
                    plsc.addupdate_scatter(hist_v, [addr], ones)
                    return gcarry

                lax.fori_loop(0, GROUPS, group_body, None)

            return carry

        lax.fori_loop(0, CHUNKS_PER_W, chunk_body, None)
        pltpu.sync_copy(hist_v, out_hbm.at[w])

    return body(inputs_flat, targets)


def _finish(partial):
    def body(p_ref, o_ref):
        x = p_ref[...]
        col = lax.broadcasted_iota(jnp.int32, (NW, 1024), 1)
        bin_ = col // 16
        ti = (bin_ // N_CATS).astype(jnp.float32)
        pj = (bin_ % N_CATS).astype(jnp.float32)
        xn = x * (1.0 / N_ROWS)
        wsq = (ti - pj) ** 2 * (1.0 / 49.0)
        po = jnp.sum((1.0 - wsq) * xn)
        s0 = jnp.sum(xn)
        s1t = jnp.sum(ti * xn)
        s2t = jnp.sum(ti * ti * xn)
        s1p = jnp.sum(pj * xn)
        s2p = jnp.sum(pj * pj * xn)
        pe = s0 * s0 - (s2t * s0 - 2.0 * s1t * s1p + s0 * s2p) * (1.0 / 49.0)
        pe = jnp.clip(pe, 0.0, 1.0 - EPS)
        qwk = jnp.where(pe >= 1.0 - EPS, 0.0, (po - pe) / (1.0 - pe + EPS))
        qwk = jnp.clip(qwk, -1.0, 1.0)
        o_ref[...] = jnp.broadcast_to(1.0 - qwk, (1, 1))

    out = pl.pallas_call(
        body, out_shape=jax.ShapeDtypeStruct((1, 1), jnp.float32))(partial)
    return out[0, 0]


def kernel(inputs, targets):
    if inputs.ndim > 2:
        inputs = inputs.reshape(-1, inputs.shape[-1])
        targets = targets.reshape(-1)
    partial = _sc_partial_hist(inputs.reshape(-1), targets.astype(jnp.int32))
    return _finish(partial)
